# baseline (speedup 1.0000x reference)
import functools

import jax
import jax.numpy as jnp
from jax import lax
from jax.experimental import pallas as pl
from jax.experimental.pallas import tpu as pltpu
from jax.experimental.pallas import tpu_sc as plsc

B = 16384
F = 26
V = 100000
D = 16
NUM = 13
OW = NUM + F * D

L = 16
NC, NS = 2, 16
NW = NC * NS
NBR = B // NW
RPW = NBR * F
SL = 128
SPC = 13
CR = SPC * SL
RB = CR // F
NSC = RPW // CR


@functools.partial(
    pl.kernel,
    out_type=jax.ShapeDtypeStruct((B * OW,), jnp.float32),
    mesh=plsc.VectorSubcoreMesh(core_axis_name="c", subcore_axis_name="s"),
    scratch_types=[
        pltpu.VMEM((RPW,), jnp.int32),
        pltpu.VMEM((NBR * NUM + L,), jnp.float32),
        pltpu.VMEM((2, CR, D), jnp.float32),
        pltpu.VMEM((2, RB * OW), jnp.float32),
        pltpu.SemaphoreType.DMA,
        pltpu.SemaphoreType.DMA,
    ],
    compiler_params=pltpu.CompilerParams(use_tc_tiling_on_sc=False),
)
def _emb_kernel(tab_hbm, idx_hbm, num_hbm, out_hbm, idx_v, num_v, gbuf, rbuf, gsem, wsem):
    wid = lax.axis_index("s") * NC + lax.axis_index("c")
    pltpu.sync_copy(idx_hbm.at[pl.ds(wid * RPW, RPW)], idx_v)
    pltpu.sync_copy(
        num_hbm.at[pl.ds(wid * NBR * NUM, NBR * NUM)],
        num_v.at[pl.ds(0, NBR * NUM)],
    )

    lane = lax.iota(jnp.int32, L)

    def fix(i, carry):
        j = i * L
        f = lax.rem(j + lane, F)
        idx_v[pl.ds(j, L)] = idx_v[pl.ds(j, L)] + f * V
        return carry

    lax.fori_loop(0, RPW // L, fix, 0)

    def fire_sc(c):
        def fire(k, carry):
            pltpu.make_async_copy(
                tab_hbm.at[idx_v.at[pl.ds((c * SPC + k) * SL, SL)]],
                gbuf.at[c % 2, pl.ds(k * SL, SL), :],
                gsem,
            ).start()
            return carry

        lax.fori_loop(0, SPC, fire, 0)

    def drain_sc(c):
        def drain(k, carry):
            pltpu.make_async_copy(
                tab_hbm.at[idx_v.at[pl.ds((c * SPC + k) * SL, SL)]],
                gbuf.at[c % 2, pl.ds(k * SL, SL), :],
                gsem,
            ).wait()
            return carry

        lax.fori_loop(0, SPC, drain, 0)

    def wb_copy(c):
        return pltpu.make_async_copy(
            rbuf.at[c % 2],
            out_hbm.at[pl.ds((wid * NBR + c * RB) * OW, RB * OW)],
            wsem,
        )

    def assemble(c):
        g = gbuf.at[c % 2]
        rb = rbuf.at[c % 2]

        def row(r, carry):
            rb[pl.ds(r * OW, L)] = num_v[pl.ds((c * RB + r) * NUM, L)]
            for f in range(F):
                rb[pl.ds(r * OW + NUM + f * D, D)] = g[r * F + f, :]
            return carry

        lax.fori_loop(0, RB, row, 0)

    fire_sc(0)

    def chunk_body(c, carry):
        @pl.when(c + 1 < NSC)
        def _():
            fire_sc(c + 1)

        drain_sc(c)

        @pl.when(c >= 2)
        def _():
            wb_copy(c - 2).wait()

        assemble(c)
        wb_copy(c).start()
        return carry

    lax.fori_loop(0, NSC, chunk_body, 0)
    wb_copy(NSC - 2).wait()
    wb_copy(NSC - 1).wait()


def kernel(indices, numerical, tables):
    out = _emb_kernel(
        tables.reshape(F * V, D),
        indices.reshape(B * F),
        numerical.reshape(B * NUM),
    )
    return out.reshape(B, OW)


# speedup vs baseline: 1.9585x; 1.9585x over previous
import functools

import jax
import jax.numpy as jnp
from jax import lax
from jax.experimental import pallas as pl
from jax.experimental.pallas import tpu as pltpu
from jax.experimental.pallas import tpu_sc as plsc

B = 16384
F = 26
V = 100000
D = 16
NUM = 13
OW = NUM + F * D

L = 16
NC, NS = 2, 16
NW = NC * NS
COLS = F * D
CPW = COLS // NW
SL = 128
NST = B // SL


@functools.partial(
    pl.kernel,
    out_type=jax.ShapeDtypeStruct((OW * B,), jnp.float32),
    mesh=plsc.VectorSubcoreMesh(core_axis_name="c", subcore_axis_name="s"),
    scratch_types=[
        pltpu.VMEM((2, B), jnp.int32),
        pltpu.VMEM((2, B), jnp.float32),
        pltpu.VMEM((B,), jnp.float32),
        pltpu.SemaphoreType.DMA,
        pltpu.SemaphoreType.DMA,
    ],
    compiler_params=pltpu.CompilerParams(use_tc_tiling_on_sc=False),
)
def _emb_kernel(tab_hbm, idx_hbm, num_hbm, out_hbm, idx2, col2, numv, gsem, wsem):
    wid = lax.axis_index("s") * NC + lax.axis_index("c")

    def prep(c):
        j = wid * CPW + c
        f = lax.div(j, D)
        pltpu.sync_copy(idx_hbm.at[pl.ds(f * B, B)], idx2.at[c % 2])
        off = j * V

        def addo(p, carry):
            idx2[c % 2, pl.ds(p * L, L)] = idx2[c % 2, pl.ds(p * L, L)] + off
            return carry

        lax.fori_loop(0, B // L, addo, 0)

    def fire(c):
        def go(k, carry):
            pltpu.make_async_copy(
                tab_hbm.at[idx2.at[c % 2, pl.ds(k * SL, SL)]],
                col2.at[c % 2, pl.ds(k * SL, SL)],
                gsem,
            ).start()
            return carry

        lax.fori_loop(0, NST, go, 0)

    def drain(c):
        def go(k, carry):
            pltpu.make_async_copy(
                tab_hbm.at[idx2.at[c % 2, pl.ds(k * SL, SL)]],
                col2.at[c % 2, pl.ds(k * SL, SL)],
                gsem,
            ).wait()
            return carry

        lax.fori_loop(0, NST, go, 0)

    def wb(c):
        j = wid * CPW + c
        return pltpu.make_async_copy(
            col2.at[c % 2],
            out_hbm.at[pl.ds((NUM + j) * B, B)],
            wsem,
        )

    prep(0)
    fire(0)

    def body(c, carry):
        @pl.when(c + 1 < CPW)
        def _():
            prep(c + 1)

            @pl.when(c >= 1)
            def _():
                wb(c - 1).wait()

            fire(c + 1)

        drain(c)
        wb(c).start()
        return carry

    lax.fori_loop(0, CPW, body, 0)
    wb(CPW - 2).wait()
    wb(CPW - 1).wait()

    @pl.when(wid < NUM)
    def _():
        pltpu.sync_copy(num_hbm.at[pl.ds(wid * B, B)], numv)
        pltpu.sync_copy(numv, out_hbm.at[pl.ds(wid * B, B)])


def kernel(indices, numerical, tables):
    tab = jnp.transpose(tables, (0, 2, 1)).reshape(F * D * V)
    idx_t = jnp.transpose(indices).reshape(F * B)
    num_t = jnp.transpose(numerical).reshape(NUM * B)
    out = _emb_kernel(tab, idx_t, num_t)
    return jnp.transpose(out.reshape(OW, B))
